# Initial kernel scaffold; baseline (speedup 1.0000x reference)
import jax
import jax.numpy as jnp
from jax.experimental import pallas as pl


def kernel(indices, emb_table, w_deep, b_deep, w_course, b_course, w_sub, b_sub):
    raise NotImplementedError("write your pallas kernel here")


# speedup vs baseline: 2.7978x; 2.7978x over previous
import functools

import jax
import jax.numpy as jnp
from jax import lax
from jax.experimental import pallas as pl
from jax.experimental.pallas import tpu as pltpu
from jax.experimental.pallas import tpu_sc as plsc

_B, _F, _V, _D = 4096, 26, 100000, 64
_FM_IN = _F * _D
_N = _B * _F
_NC, _NS = 2, 16
_NW = _NC * _NS
_ROWS_PER_W = _N // _NW
_CHUNK = 128
_NCHUNK = _ROWS_PER_W // _CHUNK


def _make_gather():
    mesh = plsc.VectorSubcoreMesh(core_axis_name="c", subcore_axis_name="s")

    @functools.partial(
        pl.kernel,
        mesh=mesh,
        compiler_params=pltpu.CompilerParams(use_tc_tiling_on_sc=False),
        out_type=jax.ShapeDtypeStruct((_N, _D), jnp.float32),
        scratch_types=[
            pltpu.VMEM((_NCHUNK, _CHUNK), jnp.int32),
            pltpu.VMEM((2, _CHUNK, _D), jnp.float32),
            pltpu.SemaphoreType.DMA,
            pltpu.SemaphoreType.DMA,
            pltpu.SemaphoreType.DMA,
            pltpu.SemaphoreType.DMA,
        ],
    )
    def gather(idx_hbm, table_hbm, out_hbm, idx_v, rows_v, g0, g1, s0, s1):
        wid = lax.axis_index("s") * _NC + lax.axis_index("c")
        base = wid * _ROWS_PER_W
        pltpu.sync_copy(idx_hbm.at[wid], idx_v)
        gsem = (g0, g1)
        ssem = (s0, s1)
        gh = [None, None]
        sh = [None, None]
        for j in range(_NCHUNK + 1):
            bi = j % 2
            if j < _NCHUNK:
                if sh[bi] is not None:
                    sh[bi].wait()
                gh[bi] = pltpu.async_copy(
                    table_hbm.at[idx_v.at[j]], rows_v.at[bi], gsem[bi])
            if j >= 1:
                pi = (j - 1) % 2
                gh[pi].wait()
                sh[pi] = pltpu.async_copy(
                    rows_v.at[pi],
                    out_hbm.at[pl.ds(base + (j - 1) * _CHUNK, _CHUNK)],
                    ssem[pi])
        sh[(_NCHUNK - 1) % 2].wait()
        if _NCHUNK >= 2:
            sh[(_NCHUNK - 2) % 2].wait()

    return gather


_make_gather = functools.cache(_make_gather)

_BB = 512


def _dense_body(x_ref, wd_ref, bd_ref, wcm_ref, wcf_ref, bc_ref,
                wsm_ref, wsf_ref, bs_ref, outc_ref, outs_ref):
    x = x_ref[...]
    deep = jnp.dot(x, wd_ref[...], preferred_element_type=jnp.float32)
    deep = jnp.maximum(deep + bd_ref[...], 0.0)
    s1 = jnp.sum(x, axis=1, keepdims=True)
    s2 = jnp.sum(x * x, axis=1, keepdims=True)
    cross = 0.5 * (s1 * s1 - s2)
    zc = (jnp.dot(deep, wcm_ref[...], preferred_element_type=jnp.float32)
          + cross * wcf_ref[...] + bc_ref[...])
    outc_ref[...] = 1.0 / (1.0 + jnp.exp(-zc))
    zs = (jnp.dot(deep, wsm_ref[...], preferred_element_type=jnp.float32)
          + cross * wsf_ref[...] + bs_ref[...])
    outs_ref[...] = 1.0 / (1.0 + jnp.exp(-zs))


def _dense(flat, w_deep, b_deep, wc_main, wc_fm, b_course, ws_main, ws_fm, b_sub):
    grid = (_B // _BB,)
    full = lambda shape: pl.BlockSpec(shape, lambda i: (0, 0))
    return pl.pallas_call(
        _dense_body,
        grid=grid,
        in_specs=[
            pl.BlockSpec((_BB, _FM_IN), lambda i: (i, 0)),
            full(w_deep.shape),
            full(b_deep.shape),
            full(wc_main.shape),
            full(wc_fm.shape),
            full(b_course.shape),
            full(ws_main.shape),
            full(ws_fm.shape),
            full(b_sub.shape),
        ],
        out_specs=[
            pl.BlockSpec((_BB, 728), lambda i: (i, 0)),
            pl.BlockSpec((_BB, 92), lambda i: (i, 0)),
        ],
        out_shape=[
            jax.ShapeDtypeStruct((_B, 728), jnp.float32),
            jax.ShapeDtypeStruct((_B, 92), jnp.float32),
        ],
    )(flat, w_deep, b_deep, wc_main, wc_fm, b_course, ws_main, ws_fm, b_sub)


def kernel(indices, emb_table, w_deep, b_deep, w_course, b_course, w_sub, b_sub):
    idx = indices.astype(jnp.int32).reshape(_NW, _NCHUNK, _CHUNK)
    rows = _make_gather()(idx, emb_table)
    flat = rows.reshape(_B, _FM_IN)
    logits_c, logits_s = _dense(
        flat,
        w_deep,
        b_deep.reshape(1, -1),
        w_course[:16],
        w_course[16:17],
        b_course.reshape(1, -1),
        w_sub[:16],
        w_sub[16:17],
        b_sub.reshape(1, -1),
    )
    return (logits_c, logits_s)


# speedup vs baseline: 2.8063x; 1.0030x over previous
import functools

import jax
import jax.numpy as jnp
from jax import lax
from jax.experimental import pallas as pl
from jax.experimental.pallas import tpu as pltpu
from jax.experimental.pallas import tpu_sc as plsc

_B, _F, _V, _D = 4096, 26, 100000, 64
_FM_IN = _F * _D
_N = _B * _F
_NC, _NS = 2, 16
_NW = _NC * _NS
_ROWS_PER_W = _N // _NW
_CHUNK = 128
_NCHUNK = _ROWS_PER_W // _CHUNK


def _make_gather():
    mesh = plsc.VectorSubcoreMesh(core_axis_name="c", subcore_axis_name="s")

    @functools.partial(
        pl.kernel,
        mesh=mesh,
        out_type=jax.ShapeDtypeStruct((_N, 2 * _D), jnp.float32),
        scratch_types=[
            pltpu.VMEM((_NCHUNK, _CHUNK), jnp.int32),
            pltpu.VMEM((2, _CHUNK, 2 * _D), jnp.float32),
            pltpu.SemaphoreType.DMA,
            pltpu.SemaphoreType.DMA,
            pltpu.SemaphoreType.DMA,
            pltpu.SemaphoreType.DMA,
        ],
    )
    def gather(idx_hbm, table_hbm, out_hbm, idx_v, rows_v, g0, g1, s0, s1):
        wid = lax.axis_index("s") * _NC + lax.axis_index("c")
        base = wid * _ROWS_PER_W
        pltpu.sync_copy(idx_hbm.at[wid], idx_v)
        gsem = (g0, g1)
        ssem = (s0, s1)
        gh = [None, None]
        sh = [None, None]
        for j in range(_NCHUNK + 1):
            bi = j % 2
            if j < _NCHUNK:
                if sh[bi] is not None:
                    sh[bi].wait()
                gh[bi] = pltpu.async_copy(
                    table_hbm.at[idx_v.at[j]], rows_v.at[bi], gsem[bi])
            if j >= 1:
                pi = (j - 1) % 2
                gh[pi].wait()
                sh[pi] = pltpu.async_copy(
                    rows_v.at[pi],
                    out_hbm.at[pl.ds(base + (j - 1) * _CHUNK, _CHUNK)],
                    ssem[pi])
        sh[(_NCHUNK - 1) % 2].wait()
        if _NCHUNK >= 2:
            sh[(_NCHUNK - 2) % 2].wait()

    return gather


_make_gather = functools.cache(_make_gather)

_BB = 512


def _dense_body(x_ref, par_ref, wd_ref, bd_ref, wcm_ref, wcf_ref, bc_ref,
                wsm_ref, wsf_ref, bs_ref, outc_ref, outs_ref):
    hi = lax.broadcasted_iota(jnp.int32, (_BB, 2 * _D), 1) >= _D
    deep = jnp.zeros((_BB, 16), dtype=jnp.float32)
    t1 = jnp.zeros((_BB, 2 * _D), dtype=jnp.float32)
    t2 = jnp.zeros((_BB, 2 * _D), dtype=jnp.float32)
    for f in range(_F):
        xf = x_ref[f]
        keep = (par_ref[f][:, None] != 0) == hi
        xm = jnp.where(keep, xf, 0.0)
        deep = deep + jnp.dot(xm, wd_ref[f],
                              preferred_element_type=jnp.float32)
        t1 = t1 + xm
        t2 = t2 + xm * xm
    deep = jnp.maximum(deep + bd_ref[...], 0.0)
    s1 = jnp.sum(t1, axis=1, keepdims=True)
    s2 = jnp.sum(t2, axis=1, keepdims=True)
    cross = 0.5 * (s1 * s1 - s2)
    zc = (jnp.dot(deep, wcm_ref[...], preferred_element_type=jnp.float32)
          + cross * wcf_ref[...] + bc_ref[...])
    outc_ref[...] = 1.0 / (1.0 + jnp.exp(-zc))
    zs = (jnp.dot(deep, wsm_ref[...], preferred_element_type=jnp.float32)
          + cross * wsf_ref[...] + bs_ref[...])
    outs_ref[...] = 1.0 / (1.0 + jnp.exp(-zs))


def _dense(x4, parity_t, wd2, b_deep, wc_main, wc_fm, b_course,
           ws_main, ws_fm, b_sub):
    grid = (_B // _BB,)
    full2 = lambda shape: pl.BlockSpec(shape, lambda i: (0, 0))
    return pl.pallas_call(
        _dense_body,
        grid=grid,
        in_specs=[
            pl.BlockSpec((_F, _BB, 2 * _D), lambda i: (0, i, 0)),
            pl.BlockSpec((_F, _BB), lambda i: (0, i)),
            pl.BlockSpec(wd2.shape, lambda i: (0, 0, 0)),
            full2(b_deep.shape),
            full2(wc_main.shape),
            full2(wc_fm.shape),
            full2(b_course.shape),
            full2(ws_main.shape),
            full2(ws_fm.shape),
            full2(b_sub.shape),
        ],
        out_specs=[
            pl.BlockSpec((_BB, 728), lambda i: (i, 0)),
            pl.BlockSpec((_BB, 92), lambda i: (i, 0)),
        ],
        out_shape=[
            jax.ShapeDtypeStruct((_B, 728), jnp.float32),
            jax.ShapeDtypeStruct((_B, 92), jnp.float32),
        ],
    )(x4, parity_t, wd2, b_deep, wc_main, wc_fm, b_course,
      ws_main, ws_fm, b_sub)


def kernel(indices, emb_table, w_deep, b_deep, w_course, b_course, w_sub, b_sub):
    idx_t = indices.astype(jnp.int32).T
    idx_pair = (idx_t >> 1).reshape(_NW, _NCHUNK, _CHUNK)
    parity_t = (idx_t & 1)
    table2 = emb_table.reshape(_V // 2, 2 * _D)
    rows = _make_gather()(idx_pair, table2)
    x4 = rows.reshape(_F, _B, 2 * _D)
    wd3 = w_deep.reshape(_F, _D, 16)
    wd2 = jnp.concatenate([wd3, wd3], axis=1)
    logits_c, logits_s = _dense(
        x4,
        parity_t,
        wd2,
        b_deep.reshape(1, -1),
        w_course[:16],
        w_course[16:17],
        b_course.reshape(1, -1),
        w_sub[:16],
        w_sub[16:17],
        b_sub.reshape(1, -1),
    )
    return (logits_c, logits_s)
